# baseline (speedup 1.0000x reference)
import jax
from jax.experimental import pallas as pl
from jax.experimental.pallas import tpu as pltpu

_ROWS = 896
_COLS = 86016
_K = 28
_CHUNK = _ROWS // _K


def _copy_kernel(x_ref, o_ref, sems):
    copies = []
    for i in range(_K):
        c = pltpu.make_async_copy(
            x_ref.at[pl.ds(i * _CHUNK, _CHUNK), :],
            o_ref.at[pl.ds(i * _CHUNK, _CHUNK), :],
            sems.at[i],
        )
        c.start()
        copies.append(c)
    for c in copies:
        c.wait()


def kernel(x, W_qkv, b_qkv):
    del W_qkv, b_qkv
    x2 = x.reshape(_ROWS, _COLS)
    out = pl.pallas_call(
        _copy_kernel,
        out_shape=jax.ShapeDtypeStruct((_ROWS, _COLS), x.dtype),
        in_specs=[pl.BlockSpec(memory_space=pl.ANY)],
        out_specs=pl.BlockSpec(memory_space=pl.ANY),
        scratch_shapes=[pltpu.SemaphoreType.DMA((_K,))],
    )(x2)
    return out.reshape(x.shape)


# speedup vs baseline: 15.3097x; 15.3097x over previous
import jax
from jax.experimental import pallas as pl

_ROWS = 896
_COLS = 86016
_BLOCK_ROWS = 32


def _copy_kernel(x_ref, o_ref):
    o_ref[...] = x_ref[...]


def kernel(x, W_qkv, b_qkv):
    del W_qkv, b_qkv
    x2 = x.reshape(_ROWS, _COLS)
    out = pl.pallas_call(
        _copy_kernel,
        out_shape=jax.ShapeDtypeStruct((_ROWS, _COLS), x.dtype),
        grid=(_ROWS // _BLOCK_ROWS,),
        in_specs=[pl.BlockSpec((_BLOCK_ROWS, _COLS), lambda i: (i, 0))],
        out_specs=pl.BlockSpec((_BLOCK_ROWS, _COLS), lambda i: (i, 0)),
    )(x2)
    return out.reshape(x.shape)


# speedup vs baseline: 51.6134x; 3.3713x over previous
import jax
from jax.experimental import pallas as pl

_B, _H, _W, _C = 4, 224, 224, 384
_BLOCK_H = 32


def _copy_kernel(x_ref, o_ref):
    o_ref[...] = x_ref[...]


def kernel(x, W_qkv, b_qkv):
    del W_qkv, b_qkv
    return pl.pallas_call(
        _copy_kernel,
        out_shape=jax.ShapeDtypeStruct((_B, _H, _W, _C), x.dtype),
        grid=(_B, _H // _BLOCK_H),
        in_specs=[pl.BlockSpec((1, _BLOCK_H, _W, _C), lambda b, i: (b, i, 0, 0))],
        out_specs=pl.BlockSpec((1, _BLOCK_H, _W, _C), lambda b, i: (b, i, 0, 0)),
    )(x)
